# Initial kernel scaffold; baseline (speedup 1.0000x reference)
import jax
import jax.numpy as jnp
from jax.experimental import pallas as pl


def kernel(output_regression, output_classification, batch_annotations, anchors, image_shape):
    raise NotImplementedError("write your pallas kernel here")


# speedup vs baseline: 3.8557x; 3.8557x over previous
import jax
import jax.numpy as jnp
from jax.experimental import pallas as pl
from jax.experimental.pallas import tpu as pltpu

_FOCAL_ALPHA = 0.25
_SIGMA_SQ = 9.0
_POS_THRESH = 0.5
_NEG_THRESH = 0.4
_EPS = 1e-4


def _retina_block(lim_ref, ann_ref, anc_ref, reg_ref, cls_ref,
                  cls_out, reg_out, npos_out):
    anc = anc_ref[0]
    ann_t = ann_ref[0]
    m = ann_t.shape[1]

    ax1 = anc[:, 0:1]
    ay1 = anc[:, 1:2]
    ax2 = anc[:, 2:3]
    ay2 = anc[:, 3:4]
    bx1 = ann_t[0:1, :]
    by1 = ann_t[1:2, :]
    bx2 = ann_t[2:3, :]
    by2 = ann_t[3:4, :]
    bcl = ann_t[4:5, :]

    iw = jnp.maximum(jnp.minimum(ax2, bx2) - jnp.maximum(ax1, bx1), 0.0)
    ih = jnp.maximum(jnp.minimum(ay2, by2) - jnp.maximum(ay1, by1), 0.0)
    inter = iw * ih
    area_a = (ax2 - ax1) * (ay2 - ay1)
    area_b = (bx2 - bx1) * (by2 - by1)
    iou = inter / jnp.maximum(area_a + area_b - inter, 1e-8)

    max_iou = jnp.max(iou, axis=1, keepdims=True)
    lane = jax.lax.broadcasted_iota(jnp.int32, iou.shape, 1)
    best = jnp.min(jnp.where(iou == max_iou, lane, m), axis=1, keepdims=True)
    oh = (lane == best).astype(jnp.float32)
    gx1 = jnp.sum(oh * bx1, axis=1, keepdims=True)
    gy1 = jnp.sum(oh * by1, axis=1, keepdims=True)
    gx2 = jnp.sum(oh * bx2, axis=1, keepdims=True)
    gy2 = jnp.sum(oh * by2, axis=1, keepdims=True)
    gcl = jnp.sum(oh * bcl, axis=1, keepdims=True).astype(jnp.int32)

    hf = lim_ref[0]
    wf = lim_ref[1]
    cx = (ax1 + ax2) * 0.5
    cy = (ay1 + ay2) * 0.5
    inside = (cx < wf) & (cy < hf)
    pos_raw = max_iou >= _POS_THRESH
    pos = pos_raw & inside
    valid = (pos_raw | (max_iou <= _NEG_THRESH)) & inside
    posf = pos.astype(jnp.float32)
    validf = valid.astype(jnp.float32)

    aw = ax2 - ax1
    ah = ay2 - ay1
    sw = 5.0 / aw
    sh = 5.0 / ah
    t = jnp.concatenate([(gx1 - ax1) * sw, (gy1 - ay1) * sh,
                         (gx2 - ax2) * sw, (gy2 - ay2) * sh], axis=1)
    diff = jnp.abs(reg_ref[0] - t)
    sl1 = jnp.where(diff < 1.0 / _SIGMA_SQ,
                    (0.5 * _SIGMA_SQ) * diff * diff,
                    diff - 0.5 / _SIGMA_SQ)
    reg_out[0, 0] = jnp.sum(sl1 * posf, axis=0, keepdims=True)

    p = jnp.clip(cls_ref[0], _EPS, 1.0 - _EPS)
    clane = jax.lax.broadcasted_iota(jnp.int32, p.shape, 1)
    isp = (clane == gcl) & pos
    s = jnp.where(isp, p, 1.0 - p)
    alpha = jnp.where(isp, _FOCAL_ALPHA, 1.0 - _FOCAL_ALPHA)
    oms = 1.0 - s
    cls_elem = alpha * (oms * oms) * (-jnp.log(s))
    cls_out[0, 0] = jnp.sum(cls_elem * validf, axis=0, keepdims=True)

    npos_out[0, 0] = jnp.sum(posf, axis=0, keepdims=True)


def _pick_block(a):
    for d in (4000, 3000, 2000, 1000, 500, 200, 100):
        if a % d == 0 and d <= a:
            return d
    return a


def kernel(output_regression, output_classification, batch_annotations,
           anchors, image_shape):
    B, A, C = output_classification.shape
    M = batch_annotations.shape[1]
    ta = _pick_block(A)
    nb = A // ta

    ann_t = jnp.transpose(batch_annotations, (0, 2, 1))
    lims = image_shape.astype(jnp.float32)

    cls_p, reg_p, np_p = pl.pallas_call(
        _retina_block,
        grid=(B, nb),
        in_specs=[
            pl.BlockSpec(memory_space=pltpu.SMEM),
            pl.BlockSpec((1, 5, M), lambda b, i: (b, 0, 0)),
            pl.BlockSpec((1, ta, 4), lambda b, i: (b, i, 0)),
            pl.BlockSpec((1, ta, 4), lambda b, i: (b, i, 0)),
            pl.BlockSpec((1, ta, C), lambda b, i: (b, i, 0)),
        ],
        out_specs=[
            pl.BlockSpec((1, 1, 1, C), lambda b, i: (b, i, 0, 0)),
            pl.BlockSpec((1, 1, 1, 4), lambda b, i: (b, i, 0, 0)),
            pl.BlockSpec((1, 1, 1, 1), lambda b, i: (b, i, 0, 0)),
        ],
        out_shape=[
            jax.ShapeDtypeStruct((B, nb, 1, C), jnp.float32),
            jax.ShapeDtypeStruct((B, nb, 1, 4), jnp.float32),
            jax.ShapeDtypeStruct((B, nb, 1, 1), jnp.float32),
        ],
        compiler_params=pltpu.CompilerParams(
            dimension_semantics=("parallel", "arbitrary"),
        ),
    )(lims, ann_t, anchors, output_regression, output_classification)

    npos = jnp.sum(np_p)
    norm = jnp.maximum(npos, 1.0)
    return (jnp.sum(cls_p) + jnp.sum(reg_p)) / norm


# speedup vs baseline: 12.6909x; 3.2915x over previous
import jax
import jax.numpy as jnp
from jax.experimental import pallas as pl
from jax.experimental.pallas import tpu as pltpu

_FOCAL_ALPHA = 0.25
_SIGMA_SQ = 9.0
_POS_THRESH = 0.5
_NEG_THRESH = 0.4
_EPS = 1e-4
_TA = 3750


def _retina_block(lim_ref, ann_ref, anc_ref, reg_ref, cls_ref,
                  loss_out, npos_out):
    anc = anc_ref[0, 0]
    ann = ann_ref[0]
    m = ann.shape[0]

    ax1 = anc[0:1, :]
    ay1 = anc[1:2, :]
    ax2 = anc[2:3, :]
    ay2 = anc[3:4, :]
    bx1 = ann[:, 0:1]
    by1 = ann[:, 1:2]
    bx2 = ann[:, 2:3]
    by2 = ann[:, 3:4]
    bcl = ann[:, 4:5]

    iw = jnp.maximum(jnp.minimum(ax2, bx2) - jnp.maximum(ax1, bx1), 0.0)
    ih = jnp.maximum(jnp.minimum(ay2, by2) - jnp.maximum(ay1, by1), 0.0)
    inter = iw * ih
    area_a = (ax2 - ax1) * (ay2 - ay1)
    area_b = (bx2 - bx1) * (by2 - by1)
    iou = inter / jnp.maximum(area_a + area_b - inter, 1e-8)

    max_iou = jnp.max(iou, axis=0, keepdims=True)
    sub = jax.lax.broadcasted_iota(jnp.int32, iou.shape, 0)
    best = jnp.min(jnp.where(iou == max_iou, sub, m), axis=0, keepdims=True)
    oh = (sub == best).astype(jnp.float32)
    gx1 = jnp.sum(oh * bx1, axis=0, keepdims=True)
    gy1 = jnp.sum(oh * by1, axis=0, keepdims=True)
    gx2 = jnp.sum(oh * bx2, axis=0, keepdims=True)
    gy2 = jnp.sum(oh * by2, axis=0, keepdims=True)
    gcl = jnp.sum(oh * bcl, axis=0, keepdims=True).astype(jnp.int32)

    hf = lim_ref[0]
    wf = lim_ref[1]
    cx = (ax1 + ax2) * 0.5
    cy = (ay1 + ay2) * 0.5
    inside = (cx < wf) & (cy < hf)
    pos_raw = max_iou >= _POS_THRESH
    pos = pos_raw & inside
    valid = (pos_raw | (max_iou <= _NEG_THRESH)) & inside
    posf = pos.astype(jnp.float32)
    validf = valid.astype(jnp.float32)

    aw = ax2 - ax1
    ah = ay2 - ay1
    sw = 5.0 / aw
    sh = 5.0 / ah
    t = jnp.concatenate([(gx1 - ax1) * sw, (gy1 - ay1) * sh,
                         (gx2 - ax2) * sw, (gy2 - ay2) * sh], axis=0)
    diff = jnp.abs(reg_ref[0, 0] - t)
    sl1 = jnp.where(diff < 1.0 / _SIGMA_SQ,
                    (0.5 * _SIGMA_SQ) * diff * diff,
                    diff - 0.5 / _SIGMA_SQ)
    reg_row = jnp.sum(sl1, axis=0, keepdims=True) * posf

    pt = jnp.transpose(cls_ref[0, 0], (1, 0))
    p = jnp.clip(pt, _EPS, 1.0 - _EPS)
    csub = jax.lax.broadcasted_iota(jnp.int32, p.shape, 0)
    isp = (csub == gcl) & pos
    s = jnp.where(isp, p, 1.0 - p)
    alpha = jnp.where(isp, _FOCAL_ALPHA, 1.0 - _FOCAL_ALPHA)
    oms = 1.0 - s
    cls_elem = alpha * (oms * oms) * (-jnp.log(s))
    cls_row = jnp.sum(cls_elem, axis=0, keepdims=True) * validf

    loss_row = cls_row + reg_row
    loss_out[0, 0] = jnp.sum(loss_row, axis=1, keepdims=True)
    npos_out[0, 0] = jnp.sum(posf, axis=1, keepdims=True)


def kernel(output_regression, output_classification, batch_annotations,
           anchors, image_shape):
    B, A, C = output_classification.shape
    M = batch_annotations.shape[1]
    ta = _TA if A % _TA == 0 else A
    nb = A // ta

    anc_r = anchors.transpose(0, 2, 1).reshape(B, 4, nb, ta).transpose(0, 2, 1, 3)
    reg_r = output_regression.transpose(0, 2, 1).reshape(B, 4, nb, ta).transpose(0, 2, 1, 3)
    cls_r = output_classification.reshape(B, nb, ta, C)
    lims = image_shape.astype(jnp.float32)

    loss_p, np_p = pl.pallas_call(
        _retina_block,
        grid=(B, nb),
        in_specs=[
            pl.BlockSpec(memory_space=pltpu.SMEM),
            pl.BlockSpec((1, M, 5), lambda b, i: (b, 0, 0)),
            pl.BlockSpec((1, 1, 4, ta), lambda b, i: (b, i, 0, 0)),
            pl.BlockSpec((1, 1, 4, ta), lambda b, i: (b, i, 0, 0)),
            pl.BlockSpec((1, 1, ta, C), lambda b, i: (b, i, 0, 0)),
        ],
        out_specs=[
            pl.BlockSpec((1, 1, 1, 1), lambda b, i: (b, i, 0, 0)),
            pl.BlockSpec((1, 1, 1, 1), lambda b, i: (b, i, 0, 0)),
        ],
        out_shape=[
            jax.ShapeDtypeStruct((B, nb, 1, 1), jnp.float32),
            jax.ShapeDtypeStruct((B, nb, 1, 1), jnp.float32),
        ],
        compiler_params=pltpu.CompilerParams(
            dimension_semantics=("parallel", "arbitrary"),
        ),
    )(lims, batch_annotations, anc_r, reg_r, cls_r)

    npos = jnp.sum(np_p)
    norm = jnp.maximum(npos, 1.0)
    return jnp.sum(loss_p) / norm
